# Initial kernel scaffold; baseline (speedup 1.0000x reference)
import jax
import jax.numpy as jnp
from jax.experimental import pallas as pl


def kernel(x, edge_index, W_l0, b_l0, W_r0, W_l1, b_l1, W_r1):
    raise NotImplementedError("write your pallas kernel here")


# speedup vs baseline: 6.9859x; 6.9859x over previous
import jax
import jax.numpy as jnp
from jax import lax
from jax.experimental import pallas as pl
from jax.experimental.pallas import tpu as pltpu
from jax.experimental.pallas import tpu_sc as plsc

N_NODES = 10000
N_EDGES = 320000
D = 128
NC, NS = 2, 16
NW = NC * NS
NPAD = 10240
RPT = NPAD // NS
DEGW = 128
K = 80
EPW = N_EDGES // NW
NCH = EPW // K

_MESH = plsc.VectorSubcoreMesh(
    core_axis_name="c", subcore_axis_name="s", num_cores=NC, num_subcores=NS)


def _segsum_body(r_hbm, srcg, dstg, z128, out_hbm, src_v, dst_v, rows_v, sem, acc_sh):
    c = lax.axis_index("c")
    s = lax.axis_index("s")
    w = c * NS + s
    pltpu.sync_copy(z128.at[pl.ds(s * RPT, RPT)], acc_sh.at[pl.ds(s * RPT, RPT)])
    pltpu.sync_copy(srcg.at[w], src_v)
    pltpu.sync_copy(dstg.at[w], dst_v)
    plsc.subcore_barrier()

    def chunk(j, carry):
        pltpu.async_copy(r_hbm.at[src_v.at[j]], rows_v, sem).wait()
        pltpu.sync_copy(rows_v, acc_sh.at[dst_v.at[j]], add=True)
        return carry

    lax.fori_loop(0, NCH, chunk, 0)
    plsc.subcore_barrier()
    base = c * NPAD + s * RPT
    pltpu.sync_copy(acc_sh.at[pl.ds(s * RPT, RPT)], out_hbm.at[pl.ds(base, RPT)])


_segsum = pl.kernel(
    _segsum_body,
    out_type=jax.ShapeDtypeStruct((NC * NPAD, D), jnp.float32),
    mesh=_MESH,
    scratch_types=(
        pltpu.VMEM((NCH, K), jnp.int32),
        pltpu.VMEM((NCH, K), jnp.int32),
        pltpu.VMEM((K, D), jnp.float32),
        pltpu.SemaphoreType.DMA,
        pltpu.VMEM_SHARED((NPAD, D), jnp.float32),
    ),
)


def _deg_body(dstg, z16, ones_hbm, deg_hbm, dst_v, ones_v, deg_sh):
    c = lax.axis_index("c")
    s = lax.axis_index("s")
    w = c * NS + s
    pltpu.sync_copy(z16.at[pl.ds(s * RPT, RPT)], deg_sh.at[pl.ds(s * RPT, RPT)])
    pltpu.sync_copy(ones_hbm, ones_v)
    pltpu.sync_copy(dstg.at[w], dst_v)
    plsc.subcore_barrier()

    def chunk(j, carry):
        pltpu.sync_copy(ones_v, deg_sh.at[dst_v.at[j]], add=True)
        return carry

    lax.fori_loop(0, NCH, chunk, 0)
    plsc.subcore_barrier()
    base = c * NPAD + s * RPT
    pltpu.sync_copy(deg_sh.at[pl.ds(s * RPT, RPT)], deg_hbm.at[pl.ds(base, RPT)])


_deg = pl.kernel(
    _deg_body,
    out_type=jax.ShapeDtypeStruct((NC * NPAD, DEGW), jnp.float32),
    mesh=_MESH,
    scratch_types=(
        pltpu.VMEM((NCH, K), jnp.int32),
        pltpu.VMEM((K, DEGW), jnp.float32),
        pltpu.VMEM_SHARED((NPAD, DEGW), jnp.float32),
    ),
)

_NB = 10
_RB = N_NODES // _NB


def _relu_body(x_ref, o_ref):
    o_ref[...] = jnp.maximum(x_ref[...], 0.0)


_relu = pl.pallas_call(
    _relu_body,
    grid=(_NB,),
    in_specs=[pl.BlockSpec((_RB, D), lambda i: (i, 0))],
    out_specs=pl.BlockSpec((_RB, D), lambda i: (i, 0)),
    out_shape=jax.ShapeDtypeStruct((N_NODES, D), jnp.float32),
)


def _make_combine(apply_relu):
    def body(p_ref, deg_ref, xin_ref, wl_ref, wr_ref, b_ref, o_ref):
        cnt = jnp.maximum(deg_ref[0, :, 0:1] + deg_ref[1, :, 0:1], 1.0)
        agg = (p_ref[0] + p_ref[1]) / cnt
        z = (jnp.dot(agg, wl_ref[...], preferred_element_type=jnp.float32)
             + jnp.dot(xin_ref[...], wr_ref[...], preferred_element_type=jnp.float32)
             + b_ref[...])
        o_ref[...] = jnp.maximum(z, 0.0) if apply_relu else z

    return pl.pallas_call(
        body,
        grid=(_NB,),
        in_specs=[
            pl.BlockSpec((NC, _RB, D), lambda i: (0, i, 0)),
            pl.BlockSpec((NC, _RB, DEGW), lambda i: (0, i, 0)),
            pl.BlockSpec((_RB, D), lambda i: (i, 0)),
            pl.BlockSpec((D, D), lambda i: (0, 0)),
            pl.BlockSpec((D, D), lambda i: (0, 0)),
            pl.BlockSpec((1, D), lambda i: (0, 0)),
        ],
        out_specs=pl.BlockSpec((_RB, D), lambda i: (i, 0)),
        out_shape=jax.ShapeDtypeStruct((N_NODES, D), jnp.float32),
    )


_combine_relu = _make_combine(True)
_combine_id = _make_combine(False)


def kernel(x, edge_index, W_l0, b_l0, W_r0, W_l1, b_l1, W_r1):
    ei = edge_index.astype(jnp.int32)
    src = ei[0].reshape(NW, NCH, K)
    dst = ei[1].reshape(NW, NCH, K)
    z128 = jnp.zeros((NPAD, D), jnp.float32)
    z16 = jnp.zeros((NPAD, DEGW), jnp.float32)
    ones = jnp.ones((K, DEGW), jnp.float32)

    r0 = _relu(x)
    deg_flat = _deg(dst, z16, ones)
    p0_flat = _segsum(r0, src, dst, z128)
    p0 = p0_flat.reshape(NC, NPAD, D)
    degp = deg_flat.reshape(NC, NPAD, DEGW)
    z1 = _combine_relu(p0, degp, x, W_l0, W_r0, b_l0.reshape(1, D))
    p1_flat = _segsum(z1, src, dst, z128)
    p1 = p1_flat.reshape(NC, NPAD, D)
    return _combine_id(p1, degp, z1, W_l1, W_r1, b_l1.reshape(1, D))
